# baseline (speedup 1.0000x reference)
import functools

import jax
import jax.numpy as jnp
from jax import lax
from jax.experimental import pallas as pl
from jax.experimental.pallas import tpu as pltpu
from jax.experimental.pallas import tpu_sc as plsc

NC = 2
NS = 16
NW = NC * NS
CHUNK = 128


def _sc_degree(dst_p, npad):
    ch = dst_p.shape[1]
    rows_per_tile = npad // NS
    mesh = plsc.VectorSubcoreMesh(core_axis_name="c", subcore_axis_name="s")

    @functools.partial(
        pl.kernel,
        mesh=mesh,
        out_type=jax.ShapeDtypeStruct((2 * npad, 128), jnp.float32),
        scratch_types=[
            pltpu.VMEM((ch, CHUNK), jnp.int32),
            pltpu.VMEM((CHUNK, 128), jnp.float32),
            pltpu.VMEM_SHARED((npad, 128), jnp.float32),
        ],
    )
    def k(dst_hbm, out_hbm, idx_v, stage_v, acc_s):
        cid = lax.axis_index("c")
        sid = lax.axis_index("s")
        wid = sid * NC + cid
        pltpu.sync_copy(dst_hbm.at[wid], idx_v)

        @pl.loop(0, CHUNK)
        def _(i):
            @pl.loop(0, 8)
            def _(k2):
                stage_v[i, pl.ds(k2 * 16, 16)] = jnp.zeros((16,), jnp.float32)

        rbase = sid * rows_per_tile

        @pl.loop(0, rows_per_tile // CHUNK)
        def _(t):
            pltpu.sync_copy(stage_v, acc_s.at[pl.ds(rbase + t * CHUNK, CHUNK)])

        plsc.subcore_barrier()

        @pl.loop(0, CHUNK)
        def _(i):
            stage_v[i, pl.ds(0, 16)] = jnp.ones((16,), jnp.float32)

        @pl.loop(0, ch)
        def _(j):
            pltpu.sync_copy(stage_v, acc_s.at[idx_v.at[j]], add=True)

        plsc.subcore_barrier()
        pltpu.sync_copy(
            acc_s.at[pl.ds(rbase, rows_per_tile)],
            out_hbm.at[pl.ds(cid * npad + rbase, rows_per_tile)],
        )

    return k(dst_p)


def _sc_segment_sum(g, src_p, dst_p, npad, d, f, sl):
    per_sub = f + sl
    rows_per_tile = npad // NS
    mesh = plsc.VectorSubcoreMesh(core_axis_name="c", subcore_axis_name="s")

    @functools.partial(
        pl.kernel,
        mesh=mesh,
        out_type=jax.ShapeDtypeStruct((2 * npad, d), jnp.float32),
        scratch_types=[
            pltpu.VMEM((f, CHUNK), jnp.int32),
            pltpu.VMEM((f, CHUNK), jnp.int32),
            pltpu.VMEM((CHUNK, d), jnp.float32),
            pltpu.VMEM_SHARED((npad, d), jnp.float32),
            pltpu.SemaphoreType.DMA,
        ],
    )
    def k(g_hbm, src_hbm, dst_hbm, out_hbm, src_v, dst_v, rows_v, acc_s, sem):
        cid = lax.axis_index("c")
        sid = lax.axis_index("s")

        @pl.loop(0, CHUNK)
        def _(i):
            @pl.loop(0, d // 16)
            def _(k2):
                rows_v[i, pl.ds(k2 * 16, 16)] = jnp.zeros((16,), jnp.float32)

        rbase = sid * rows_per_tile

        @pl.loop(0, rows_per_tile // CHUNK)
        def _(t):
            pltpu.sync_copy(rows_v, acc_s.at[pl.ds(rbase + t * CHUNK, CHUNK)])

        plsc.subcore_barrier()

        @pl.when(cid == 0)
        def _():
            base = sid * per_sub
            pltpu.sync_copy(src_hbm.at[pl.ds(base, f)], src_v)
            pltpu.sync_copy(dst_hbm.at[pl.ds(base, f)], dst_v)

            @pl.loop(0, f)
            def _(j):
                pltpu.async_copy(g_hbm.at[src_v.at[j]], rows_v, sem).wait()
                pltpu.sync_copy(rows_v, acc_s.at[dst_v.at[j]], add=True)

        @pl.when(cid == 1)
        def _():
            base = sid * per_sub + f
            pltpu.sync_copy(src_hbm.at[pl.ds(base, sl)],
                            src_v.at[pl.ds(0, sl)])
            pltpu.sync_copy(dst_hbm.at[pl.ds(base, sl)],
                            dst_v.at[pl.ds(0, sl)])

            @pl.loop(0, sl)
            def _(j):
                pltpu.async_copy(g_hbm.at[src_v.at[j]], rows_v, sem).wait()
                pltpu.sync_copy(rows_v, acc_s.at[dst_v.at[j]], add=True)

        plsc.subcore_barrier()
        pltpu.sync_copy(
            acc_s.at[pl.ds(rbase, rows_per_tile)],
            out_hbm.at[pl.ds(cid * npad + rbase, rows_per_tile)],
        )

    return k(g, src_p, dst_p)


def _dinv(d0_r, d1_r):
    return lax.rsqrt(d0_r[:, 0:1] + d1_r[:, 0:1] + 1.0)


def _tc_layer1(x, w1, d0, d1, blk):
    n, din = x.shape
    dh = w1.shape[1]

    def body(x_r, w_r, d0_r, d1_r, o_r):
        o_r[...] = (
            jnp.dot(x_r[...], w_r[...], preferred_element_type=jnp.float32)
            * _dinv(d0_r, d1_r)
        )

    return pl.pallas_call(
        body,
        grid=(n // blk,),
        in_specs=[
            pl.BlockSpec((blk, din), lambda i: (i, 0)),
            pl.BlockSpec((din, dh), lambda i: (0, 0)),
            pl.BlockSpec((blk, 128), lambda i: (i, 0)),
            pl.BlockSpec((blk, 128), lambda i: (i, 0)),
        ],
        out_specs=pl.BlockSpec((blk, dh), lambda i: (i, 0)),
        out_shape=jax.ShapeDtypeStruct((n, dh), jnp.float32),
    )(x, w1, d0, d1)


def _tc_layer2(a0, a1, g1, d0, d1, b1, w2, blk):
    n, dh = g1.shape
    do = w2.shape[1]

    def body(a0_r, a1_r, g_r, d0_r, d1_r, b_r, w_r, o_r):
        dinv = _dinv(d0_r, d1_r)
        h = (a0_r[...] + a1_r[...] + g_r[...]) * dinv + b_r[...]
        h = jnp.maximum(h, 0.0)
        o_r[...] = (
            jnp.dot(h, w_r[...], preferred_element_type=jnp.float32) * dinv
        )

    return pl.pallas_call(
        body,
        grid=(n // blk,),
        in_specs=[
            pl.BlockSpec((blk, dh), lambda i: (i, 0)),
            pl.BlockSpec((blk, dh), lambda i: (i, 0)),
            pl.BlockSpec((blk, dh), lambda i: (i, 0)),
            pl.BlockSpec((blk, 128), lambda i: (i, 0)),
            pl.BlockSpec((blk, 128), lambda i: (i, 0)),
            pl.BlockSpec((1, dh), lambda i: (0, 0)),
            pl.BlockSpec((dh, do), lambda i: (0, 0)),
        ],
        out_specs=pl.BlockSpec((blk, do), lambda i: (i, 0)),
        out_shape=jax.ShapeDtypeStruct((n, do), jnp.float32),
    )(a0, a1, g1, d0, d1, b1, w2)


def _tc_final(a0, a1, g2, d0, d1, b2, blk):
    n, dp = g2.shape
    do = b2.shape[1]

    def body(a0_r, a1_r, g_r, d0_r, d1_r, b_r, o_r):
        s = (pl.ds(0, blk), pl.ds(0, do))
        m = (a0_r[s] + a1_r[s] + g_r[s]) * _dinv(d0_r, d1_r) + b_r[...]
        mx = jnp.max(m, axis=1, keepdims=True)
        e = jnp.exp(m - mx)
        lse = jnp.log(jnp.sum(e, axis=1, keepdims=True)) + mx
        o_r[...] = m - lse

    return pl.pallas_call(
        body,
        grid=(n // blk,),
        in_specs=[
            pl.BlockSpec((blk, dp), lambda i: (i, 0)),
            pl.BlockSpec((blk, dp), lambda i: (i, 0)),
            pl.BlockSpec((blk, dp), lambda i: (i, 0)),
            pl.BlockSpec((blk, 128), lambda i: (i, 0)),
            pl.BlockSpec((blk, 128), lambda i: (i, 0)),
            pl.BlockSpec((1, do), lambda i: (0, 0)),
        ],
        out_specs=pl.BlockSpec((blk, do), lambda i: (i, 0)),
        out_shape=jax.ShapeDtypeStruct((n, do), jnp.float32),
    )(a0, a1, g2, d0, d1, b2)


def kernel(x, edge_index, W1, b1, W2, b2):
    n = x.shape[0]
    e = edge_index.shape[1]
    npad = -(-n // (NS * CHUNK)) * NS * CHUNK
    blk = 1000 if n % 1000 == 0 else 8

    src = edge_index[0].astype(jnp.int32)
    dst = edge_index[1].astype(jnp.int32)
    ch = -(-e // (NW * CHUNK))
    epad = NW * ch * CHUNK
    src_p = jnp.concatenate(
        [src, jnp.zeros((epad - e,), jnp.int32)]).reshape(NW, ch, CHUNK)
    dst_p = jnp.concatenate(
        [dst, jnp.full((epad - e,), n, jnp.int32)]).reshape(NW, ch, CHUNK)

    per_sub = 16 * (-(-2 * ch // 16))
    f = 8 * int(round(0.65 * per_sub / 8))
    sl = per_sub - f
    epad2 = NS * per_sub * CHUNK
    src_q = jnp.concatenate(
        [src, jnp.zeros((epad2 - e,), jnp.int32)]).reshape(NS * per_sub, CHUNK)
    dst_q = jnp.concatenate(
        [dst, jnp.full((epad2 - e,), n, jnp.int32)]).reshape(NS * per_sub, CHUNK)

    degs = _sc_degree(dst_p, npad)
    d0, d1 = degs[:n], degs[npad:npad + n]

    g1 = _tc_layer1(x, W1, d0, d1, blk)
    acc1 = _sc_segment_sum(g1, src_q, dst_q, npad, g1.shape[1], f, sl)
    a0, a1 = acc1[:n], acc1[npad:npad + n]

    do = W2.shape[1]
    w2p = jnp.concatenate(
        [W2, jnp.zeros((W2.shape[0], 128 - do), jnp.float32)], axis=1)
    g2 = _tc_layer2(a0, a1, g1, d0, d1, b1.reshape(1, -1), w2p, blk)
    acc2 = _sc_segment_sum(g2, src_q, dst_q, npad, g2.shape[1], f, sl)
    c0, c1 = acc2[:n], acc2[npad:npad + n]

    return _tc_final(c0, c1, g2, d0, d1, b2.reshape(1, -1), blk)


# speedup vs baseline: 1.3731x; 1.3731x over previous
import functools

import jax
import jax.numpy as jnp
from jax import lax
from jax.experimental import pallas as pl
from jax.experimental.pallas import tpu as pltpu
from jax.experimental.pallas import tpu_sc as plsc

NC = 2
NS = 16
NW = NC * NS
CHUNK = 128


def _sc_degree(dst_p, npad):
    ch = dst_p.shape[1]
    rows_per_tile = npad // NS
    mesh = plsc.VectorSubcoreMesh(core_axis_name="c", subcore_axis_name="s")

    @functools.partial(
        pl.kernel,
        mesh=mesh,
        out_type=jax.ShapeDtypeStruct((2 * npad, 128), jnp.float32),
        scratch_types=[
            pltpu.VMEM((ch, CHUNK), jnp.int32),
            pltpu.VMEM((CHUNK, 128), jnp.float32),
            pltpu.VMEM_SHARED((npad, 128), jnp.float32),
        ],
    )
    def k(dst_hbm, out_hbm, idx_v, stage_v, acc_s):
        cid = lax.axis_index("c")
        sid = lax.axis_index("s")
        wid = sid * NC + cid
        pltpu.sync_copy(dst_hbm.at[wid], idx_v)

        @pl.loop(0, CHUNK)
        def _(i):
            @pl.loop(0, 8)
            def _(k2):
                stage_v[i, pl.ds(k2 * 16, 16)] = jnp.zeros((16,), jnp.float32)

        rbase = sid * rows_per_tile

        @pl.loop(0, rows_per_tile // CHUNK)
        def _(t):
            pltpu.sync_copy(stage_v, acc_s.at[pl.ds(rbase + t * CHUNK, CHUNK)])

        plsc.subcore_barrier()

        @pl.loop(0, CHUNK)
        def _(i):
            stage_v[i, pl.ds(0, 16)] = jnp.ones((16,), jnp.float32)

        @pl.loop(0, ch)
        def _(j):
            pltpu.sync_copy(stage_v, acc_s.at[idx_v.at[j]], add=True)

        plsc.subcore_barrier()
        pltpu.sync_copy(
            acc_s.at[pl.ds(rbase, rows_per_tile)],
            out_hbm.at[pl.ds(cid * npad + rbase, rows_per_tile)],
        )

    return k(dst_p)


def _sc_segment_sum(ga, gb, src_p, dst_p, npad, d):
    ch = src_p.shape[1]
    rows_per_tile = npad // NS
    mesh = plsc.VectorSubcoreMesh(core_axis_name="c", subcore_axis_name="s")

    @functools.partial(
        pl.kernel,
        mesh=mesh,
        out_type=jax.ShapeDtypeStruct((2 * npad, d), jnp.float32),
        scratch_types=[
            pltpu.VMEM((ch, CHUNK), jnp.int32),
            pltpu.VMEM((ch, CHUNK), jnp.int32),
            pltpu.VMEM((CHUNK, d), jnp.float32),
            pltpu.VMEM_SHARED((npad, d), jnp.float32),
            pltpu.SemaphoreType.DMA,
        ],
    )
    def k(ga_hbm, gb_hbm, src_hbm, dst_hbm, out_hbm, src_v, dst_v, rows_v,
          acc_s, sem):
        cid = lax.axis_index("c")
        sid = lax.axis_index("s")
        wid = sid * NC + cid
        pltpu.sync_copy(src_hbm.at[wid], src_v)
        pltpu.sync_copy(dst_hbm.at[wid], dst_v)

        @pl.loop(0, CHUNK)
        def _(i):
            @pl.loop(0, d // 16)
            def _(k2):
                rows_v[i, pl.ds(k2 * 16, 16)] = jnp.zeros((16,), jnp.float32)

        rbase = sid * rows_per_tile

        @pl.loop(0, rows_per_tile // CHUNK)
        def _(t):
            pltpu.sync_copy(rows_v, acc_s.at[pl.ds(rbase + t * CHUNK, CHUNK)])

        plsc.subcore_barrier()

        @pl.when(cid == 0)
        def _():
            @pl.loop(0, ch)
            def _(j):
                pltpu.async_copy(ga_hbm.at[src_v.at[j]], rows_v, sem).wait()
                pltpu.sync_copy(rows_v, acc_s.at[dst_v.at[j]], add=True)

        @pl.when(cid == 1)
        def _():
            @pl.loop(0, ch)
            def _(j):
                pltpu.async_copy(gb_hbm.at[src_v.at[j]], rows_v, sem).wait()
                pltpu.sync_copy(rows_v, acc_s.at[dst_v.at[j]], add=True)

        plsc.subcore_barrier()
        pltpu.sync_copy(
            acc_s.at[pl.ds(rbase, rows_per_tile)],
            out_hbm.at[pl.ds(cid * npad + rbase, rows_per_tile)],
        )

    return k(ga, gb, src_p, dst_p)


def _dinv(d0_r, d1_r):
    return lax.rsqrt(d0_r[:, 0:1] + d1_r[:, 0:1] + 1.0)


def _tc_layer1(x, w1, d0, d1, blk):
    n, din = x.shape
    dh = w1.shape[1]

    def body(x_r, w_r, d0_r, d1_r, o_r, o2_r):
        v = (
            jnp.dot(x_r[...], w_r[...], preferred_element_type=jnp.float32)
            * _dinv(d0_r, d1_r)
        )
        o_r[...] = v
        o2_r[...] = v

    return pl.pallas_call(
        body,
        grid=(n // blk,),
        in_specs=[
            pl.BlockSpec((blk, din), lambda i: (i, 0)),
            pl.BlockSpec((din, dh), lambda i: (0, 0)),
            pl.BlockSpec((blk, 128), lambda i: (i, 0)),
            pl.BlockSpec((blk, 128), lambda i: (i, 0)),
        ],
        out_specs=[pl.BlockSpec((blk, dh), lambda i: (i, 0)),
                   pl.BlockSpec((blk, dh), lambda i: (i, 0))],
        out_shape=[jax.ShapeDtypeStruct((n, dh), jnp.float32),
                   jax.ShapeDtypeStruct((n, dh), jnp.float32)],
    )(x, w1, d0, d1)


def _tc_layer2(a0, a1, g1, d0, d1, b1, w2, blk):
    n, dh = g1.shape
    do = w2.shape[1]

    def body(a0_r, a1_r, g_r, d0_r, d1_r, b_r, w_r, o_r, o2_r):
        dinv = _dinv(d0_r, d1_r)
        h = (a0_r[...] + a1_r[...] + g_r[...]) * dinv + b_r[...]
        h = jnp.maximum(h, 0.0)
        v = jnp.dot(h, w_r[...], preferred_element_type=jnp.float32) * dinv
        o_r[...] = v
        o2_r[...] = v

    return pl.pallas_call(
        body,
        grid=(n // blk,),
        in_specs=[
            pl.BlockSpec((blk, dh), lambda i: (i, 0)),
            pl.BlockSpec((blk, dh), lambda i: (i, 0)),
            pl.BlockSpec((blk, dh), lambda i: (i, 0)),
            pl.BlockSpec((blk, 128), lambda i: (i, 0)),
            pl.BlockSpec((blk, 128), lambda i: (i, 0)),
            pl.BlockSpec((1, dh), lambda i: (0, 0)),
            pl.BlockSpec((dh, do), lambda i: (0, 0)),
        ],
        out_specs=[pl.BlockSpec((blk, do), lambda i: (i, 0)),
                   pl.BlockSpec((blk, do), lambda i: (i, 0))],
        out_shape=[jax.ShapeDtypeStruct((n, do), jnp.float32),
                   jax.ShapeDtypeStruct((n, do), jnp.float32)],
    )(a0, a1, g1, d0, d1, b1, w2)


def _tc_final(a0, a1, g2, d0, d1, b2, blk):
    n, dp = g2.shape
    do = b2.shape[1]

    def body(a0_r, a1_r, g_r, d0_r, d1_r, b_r, o_r):
        s = (pl.ds(0, blk), pl.ds(0, do))
        m = (a0_r[s] + a1_r[s] + g_r[s]) * _dinv(d0_r, d1_r) + b_r[...]
        mx = jnp.max(m, axis=1, keepdims=True)
        e = jnp.exp(m - mx)
        lse = jnp.log(jnp.sum(e, axis=1, keepdims=True)) + mx
        o_r[...] = m - lse

    return pl.pallas_call(
        body,
        grid=(n // blk,),
        in_specs=[
            pl.BlockSpec((blk, dp), lambda i: (i, 0)),
            pl.BlockSpec((blk, dp), lambda i: (i, 0)),
            pl.BlockSpec((blk, dp), lambda i: (i, 0)),
            pl.BlockSpec((blk, 128), lambda i: (i, 0)),
            pl.BlockSpec((blk, 128), lambda i: (i, 0)),
            pl.BlockSpec((1, do), lambda i: (0, 0)),
        ],
        out_specs=pl.BlockSpec((blk, do), lambda i: (i, 0)),
        out_shape=jax.ShapeDtypeStruct((n, do), jnp.float32),
    )(a0, a1, g2, d0, d1, b2)


def kernel(x, edge_index, W1, b1, W2, b2):
    n = x.shape[0]
    e = edge_index.shape[1]
    npad = -(-n // (NS * CHUNK)) * NS * CHUNK
    blk = 1000 if n % 1000 == 0 else 8

    src = edge_index[0].astype(jnp.int32)
    dst = edge_index[1].astype(jnp.int32)
    ch = -(-e // (NW * CHUNK))
    epad = NW * ch * CHUNK
    src_p = jnp.concatenate(
        [src, jnp.zeros((epad - e,), jnp.int32)]).reshape(NW, ch, CHUNK)
    dst_p = jnp.concatenate(
        [dst, jnp.full((epad - e,), n, jnp.int32)]).reshape(NW, ch, CHUNK)

    degs = _sc_degree(dst_p, npad)
    d0, d1 = degs[:n], degs[npad:npad + n]

    g1, g1b = _tc_layer1(x, W1, d0, d1, blk)
    acc1 = _sc_segment_sum(g1, g1b, src_p, dst_p, npad, g1.shape[1])
    a0, a1 = acc1[:n], acc1[npad:npad + n]

    do = W2.shape[1]
    w2p = jnp.concatenate(
        [W2, jnp.zeros((W2.shape[0], 128 - do), jnp.float32)], axis=1)
    g2, g2b = _tc_layer2(a0, a1, g1, d0, d1, b1.reshape(1, -1), w2p, blk)
    acc2 = _sc_segment_sum(g2, g2b, src_p, dst_p, npad, g2.shape[1])
    c0, c1 = acc2[:n], acc2[npad:npad + n]

    return _tc_final(c0, c1, g2, d0, d1, b2.reshape(1, -1), blk)
